# baseline (speedup 1.0000x reference)
import jax
import jax.numpy as jnp
from jax.experimental import pallas as pl
from jax.experimental.pallas import tpu as pltpu

_N, _D, _E, _K = 8192, 2048, 16, 2
_EP = 128
_TN = 512
_NT = _N // _TN
_CAST = 8
_CROWS = _D // _CAST


def _moe_body(x_ref, wr_ref, br_ref, we_ref, be_ref, out_ref, loss_ref,
              web_ref, a_ref, w0_ref, w1_ref):
    pid = pl.program_id(0)

    @pl.when(pid < _CAST)
    def _cast_phase():
        web_ref[:, pl.ds(jnp.minimum(pid, _CAST - 1) * _CROWS, _CROWS), :] = (
            we_ref[...].astype(jnp.bfloat16))

    @pl.when(pid == 0)
    def _init_loss():
        loss_ref[...] = jnp.zeros_like(loss_ref)

    @pl.when(pid >= _CAST)
    def _compute_phase():
        t = pid - _CAST
        par = t % 2

        xb = x_ref[...].astype(jnp.bfloat16)

        logits = jax.lax.dot_general(
            xb, wr_ref[...], (((1,), (1,)), ((), ())),
            preferred_element_type=jnp.float32)
        logits = logits + br_ref[...]
        m = jnp.max(logits, axis=-1, keepdims=True)
        e = jnp.exp(logits - m)
        s = jnp.sum(e, axis=-1, keepdims=True)
        m1 = jnp.max(e, axis=-1, keepdims=True)
        lane = jax.lax.broadcasted_iota(jnp.int32, (_TN, _EP), 1)
        first_idx = jnp.min(jnp.where(e == m1, lane, _EP), axis=-1,
                            keepdims=True)
        e_masked = jnp.where(lane == first_idx, -jnp.inf, e)
        m2 = jnp.max(e_masked, axis=-1, keepdims=True)
        tot = m1 + m2
        w0_ref[par] = m1 / tot
        w1_ref[par] = m2 / tot

        factor = jnp.where(t < _NT, 1.0 / _N, 0.0)
        loss_ref[...] += jnp.sum(tot / s, keepdims=True) * factor

        a_ref[par, 0] = jax.lax.dot_general(
            xb, web_ref[0], (((1,), (1,)), ((), ())),
            preferred_element_type=jnp.float32).astype(jnp.bfloat16)
        a_ref[par, 1] = jax.lax.dot_general(
            xb, web_ref[1], (((1,), (1,)), ((), ())),
            preferred_element_type=jnp.float32).astype(jnp.bfloat16)

        parc = jnp.maximum(t - 1, 0) % 2
        w0 = w0_ref[parc]
        w1 = w1_ref[parc]
        out_ref[...] = (w0 * a_ref[parc, 0].astype(jnp.float32)
                        + w1 * a_ref[parc, 1].astype(jnp.float32)
                        + w0 * be_ref[0:1, :] + w1 * be_ref[1:2, :])


def kernel(x, Wr, br, We, be):
    wr_p = jnp.zeros((_EP, _D), jnp.bfloat16).at[:_E].set(Wr.astype(jnp.bfloat16))
    br_p = jnp.full((1, _EP), -1e30, jnp.float32).at[0, :_E].set(br)

    grid = _CAST + _NT + 1
    out, loss = pl.pallas_call(
        _moe_body,
        grid=(grid,),
        in_specs=[
            pl.BlockSpec((_TN, _D),
                         lambda n: (jnp.minimum(jnp.maximum(n - _CAST, 0),
                                                _NT - 1), 0)),
            pl.BlockSpec((_EP, _D), lambda n: (0, 0)),
            pl.BlockSpec((1, _EP), lambda n: (0, 0)),
            pl.BlockSpec((_K, _CROWS, _D),
                         lambda n: (0, jnp.minimum(n, _CAST - 1), 0)),
            pl.BlockSpec((_K, _D), lambda n: (0, 0)),
        ],
        out_specs=[
            pl.BlockSpec((_TN, _D),
                         lambda n: (jnp.maximum(n - _CAST - 1, 0), 0)),
            pl.BlockSpec((1, 1), lambda n: (0, 0)),
        ],
        out_shape=[
            jax.ShapeDtypeStruct((_N, _D), jnp.float32),
            jax.ShapeDtypeStruct((1, 1), jnp.float32),
        ],
        scratch_shapes=[
            pltpu.VMEM((_K, _D, _D), jnp.bfloat16),
            pltpu.VMEM((2, 2, _TN, _D), jnp.bfloat16),
            pltpu.VMEM((2, _TN, 1), jnp.float32),
            pltpu.VMEM((2, _TN, 1), jnp.float32),
        ],
    )(x, wr_p, br_p, We, be)
    return out, loss[0, 0]


# speedup vs baseline: 1.1619x; 1.1619x over previous
import jax
import jax.numpy as jnp
from jax.experimental import pallas as pl
from jax.experimental.pallas import tpu as pltpu

_N, _D, _E, _K = 8192, 2048, 16, 2
_EP = 128
_TN = 512
_CAST = 4
_CROWS = _D // _CAST


def _moe_body(x_ref, wr_ref, br_ref, we_ref, be_ref, out_ref, loss_ref,
              web_ref):
    pid = pl.program_id(0)

    @pl.when(pid < _CAST)
    def _cast_phase():
        web_ref[:, pl.ds(jnp.minimum(pid, _CAST - 1) * _CROWS, _CROWS), :] = (
            we_ref[...].astype(jnp.bfloat16))

    @pl.when(pid == 0)
    def _init_loss():
        loss_ref[...] = jnp.zeros_like(loss_ref)

    @pl.when(pid >= _CAST)
    def _compute_phase():
        xb = x_ref[...].astype(jnp.bfloat16)

        logits = jax.lax.dot_general(
            xb, wr_ref[...], (((1,), (1,)), ((), ())),
            preferred_element_type=jnp.float32)
        logits = logits + br_ref[...]
        m = jnp.max(logits, axis=-1, keepdims=True)
        e = jnp.exp(logits - m)
        s = jnp.sum(e, axis=-1, keepdims=True)
        m1 = jnp.max(e, axis=-1, keepdims=True)
        lane = jax.lax.broadcasted_iota(jnp.int32, (_TN, _EP), 1)
        first_idx = jnp.min(jnp.where(e == m1, lane, _EP), axis=-1,
                            keepdims=True)
        e_masked = jnp.where(lane == first_idx, -jnp.inf, e)
        m2 = jnp.max(e_masked, axis=-1, keepdims=True)
        tot = m1 + m2
        w0 = m1 / tot
        w1 = m2 / tot

        loss_ref[...] += jnp.sum(tot / s, keepdims=True) * (1.0 / _N)

        a0 = jax.lax.dot_general(
            xb, web_ref[0], (((1,), (1,)), ((), ())),
            preferred_element_type=jnp.float32)
        out_ref[...] = w0 * a0 + (w0 * be_ref[0:1, :] + w1 * be_ref[1:2, :])
        a1 = jax.lax.dot_general(
            xb, web_ref[1], (((1,), (1,)), ((), ())),
            preferred_element_type=jnp.float32)
        out_ref[...] += w1 * a1


def kernel(x, Wr, br, We, be):
    wr_p = jnp.zeros((_EP, _D), jnp.bfloat16).at[:_E].set(Wr.astype(jnp.bfloat16))
    br_p = jnp.full((1, _EP), -1e30, jnp.float32).at[0, :_E].set(br)

    grid = _CAST + _N // _TN
    out, loss = pl.pallas_call(
        _moe_body,
        grid=(grid,),
        in_specs=[
            pl.BlockSpec((_TN, _D),
                         lambda n: (jnp.maximum(n - _CAST, 0), 0)),
            pl.BlockSpec((_EP, _D), lambda n: (0, 0)),
            pl.BlockSpec((1, _EP), lambda n: (0, 0)),
            pl.BlockSpec((_K, _CROWS, _D),
                         lambda n: (0, jnp.minimum(n, _CAST - 1), 0)),
            pl.BlockSpec((_K, _D), lambda n: (0, 0)),
        ],
        out_specs=[
            pl.BlockSpec((_TN, _D),
                         lambda n: (jnp.maximum(n - _CAST, 0), 0)),
            pl.BlockSpec((1, 1), lambda n: (0, 0)),
        ],
        out_shape=[
            jax.ShapeDtypeStruct((_N, _D), jnp.float32),
            jax.ShapeDtypeStruct((1, 1), jnp.float32),
        ],
        scratch_shapes=[pltpu.VMEM((_K, _D, _D), jnp.bfloat16)],
    )(x, wr_p, br_p, We, be)
    return out, loss[0, 0]
